# baseline (speedup 1.0000x reference)
import jax
import jax.numpy as jnp
from jax.experimental import pallas as pl
from jax.experimental.pallas import tpu as pltpu


def _feat_kernel(x_ref, w_ref, o_ref, col_ref):
    p = jnp.dot(
        x_ref[...], w_ref[...], preferred_element_type=jnp.float32
    ).astype(jnp.bfloat16)
    o_ref[...] = p
    part = jnp.sum(p.astype(jnp.float32), axis=0, keepdims=True)

    @pl.when(pl.program_id(0) == 0)
    def _():
        col_ref[...] = jnp.zeros_like(col_ref)

    col_ref[...] += part


def _layer0_kernel(g_ref, p_ref, pcol_ref, w_ref, gq_ref, o_ref, col_ref):
    t = g_ref[...] - 0.5
    gq_ref[...] = t.astype(jnp.float8_e4m3fn)
    h = jnp.dot(
        t.astype(jnp.bfloat16), p_ref[...], preferred_element_type=jnp.float32
    )
    h = h + 0.5 * pcol_ref[...]
    h = jnp.maximum(h, 0.0).astype(jnp.bfloat16)
    p = jnp.dot(
        h, w_ref[...], preferred_element_type=jnp.float32
    ).astype(jnp.bfloat16)
    o_ref[...] = p
    part = jnp.sum(p.astype(jnp.float32), axis=0, keepdims=True)

    @pl.when(pl.program_id(0) == 0)
    def _():
        col_ref[...] = jnp.zeros_like(col_ref)

    col_ref[...] += part


def _quant_hilo(p_ref, phi_ref, plo_ref, pcol_ref, s_ref):
    p = p_ref[...].astype(jnp.float32)
    s = 440.0 / jnp.maximum(jnp.max(jnp.abs(p)), 1e-30)
    ps = p * s
    phi = ps.astype(jnp.float8_e4m3fn)
    plo = ((ps - phi.astype(jnp.float32)) * 16.0).astype(jnp.float8_e4m3fn)
    phi_ref[...] = phi
    plo_ref[...] = plo
    col = jnp.sum(phi.astype(jnp.float32), axis=0, keepdims=True)
    col += jnp.sum(plo.astype(jnp.float32), axis=0, keepdims=True) * (1.0 / 16.0)
    pcol_ref[...] = col * (1.0 / s)
    s_ref[...] = jnp.full((1, 1), s, jnp.float32)


def _spmm_hilo(g_ref, phi_ref, plo_ref, pcol_ref, s_ref):
    s = s_ref[0, 0]
    gq = g_ref[...]
    acc = jnp.dot(gq, phi_ref[...], preferred_element_type=jnp.float32)
    acc += jnp.dot(
        gq, plo_ref[...], preferred_element_type=jnp.float32
    ) * (1.0 / 16.0)
    return acc * (1.0 / s) + 0.5 * pcol_ref[...]


def _layer1_kernel(
    g_ref, p_ref, w_ref, o_ref, col_ref, phi_ref, plo_ref, pcol_ref, s_ref
):
    @pl.when(pl.program_id(0) == 0)
    def _():
        _quant_hilo(p_ref, phi_ref, plo_ref, pcol_ref, s_ref)

    h = _spmm_hilo(g_ref, phi_ref, plo_ref, pcol_ref, s_ref)
    h = jnp.maximum(h, 0.0).astype(jnp.bfloat16)
    p = jnp.dot(
        h, w_ref[...], preferred_element_type=jnp.float32
    ).astype(jnp.bfloat16)
    o_ref[...] = p
    part = jnp.sum(p.astype(jnp.float32), axis=0, keepdims=True)

    @pl.when(pl.program_id(0) == 0)
    def _():
        col_ref[...] = jnp.zeros_like(col_ref)

    col_ref[...] += part


def _layer2_kernel(g_ref, p_ref, o_ref, phi_ref, plo_ref, pcol_ref, s_ref):
    @pl.when(pl.program_id(0) == 0)
    def _():
        _quant_hilo(p_ref, phi_ref, plo_ref, pcol_ref, s_ref)

    o_ref[...] = _spmm_hilo(g_ref, phi_ref, plo_ref, pcol_ref, s_ref)


def kernel(g, inputs, W0, W1, W2):
    n, _ = g.shape
    hid = W0.shape[1]
    out_dim = W2.shape[1]

    bi0 = 400 if n % 400 == 0 else 8
    bi = 1000 if n % 1000 == 0 else (400 if n % 400 == 0 else 8)

    w1b = W1.astype(jnp.bfloat16)
    w2b = W2.astype(jnp.bfloat16)
    f8 = jnp.float8_e4m3fn

    p0, col0 = pl.pallas_call(
        _feat_kernel,
        grid=(n // bi,),
        in_specs=[
            pl.BlockSpec((bi, inputs.shape[1]), lambda i: (i, 0)),
            pl.BlockSpec((inputs.shape[1], hid), lambda i: (0, 0)),
        ],
        out_specs=[
            pl.BlockSpec((bi, hid), lambda i: (i, 0)),
            pl.BlockSpec((1, hid), lambda i: (0, 0)),
        ],
        out_shape=[
            jax.ShapeDtypeStruct((n, hid), jnp.bfloat16),
            jax.ShapeDtypeStruct((1, hid), jnp.float32),
        ],
        compiler_params=pltpu.CompilerParams(
            dimension_semantics=("arbitrary",),
        ),
    )(inputs, W0)

    gq, p1, _col1 = pl.pallas_call(
        _layer0_kernel,
        grid=(n // bi0,),
        in_specs=[
            pl.BlockSpec((bi0, n), lambda i: (i, 0)),
            pl.BlockSpec((n, hid), lambda i: (0, 0)),
            pl.BlockSpec((1, hid), lambda i: (0, 0)),
            pl.BlockSpec((hid, hid), lambda i: (0, 0)),
        ],
        out_specs=[
            pl.BlockSpec((bi0, n), lambda i: (i, 0)),
            pl.BlockSpec((bi0, hid), lambda i: (i, 0)),
            pl.BlockSpec((1, hid), lambda i: (0, 0)),
        ],
        out_shape=[
            jax.ShapeDtypeStruct((n, n), f8),
            jax.ShapeDtypeStruct((n, hid), jnp.bfloat16),
            jax.ShapeDtypeStruct((1, hid), jnp.float32),
        ],
        compiler_params=pltpu.CompilerParams(
            dimension_semantics=("arbitrary",),
        ),
    )(g, p0, col0, w1b)

    p2, _col2 = pl.pallas_call(
        _layer1_kernel,
        grid=(n // bi,),
        in_specs=[
            pl.BlockSpec((bi, n), lambda i: (i, 0)),
            pl.BlockSpec((n, hid), lambda i: (0, 0)),
            pl.BlockSpec((hid, out_dim), lambda i: (0, 0)),
        ],
        out_specs=[
            pl.BlockSpec((bi, out_dim), lambda i: (i, 0)),
            pl.BlockSpec((1, out_dim), lambda i: (0, 0)),
        ],
        out_shape=[
            jax.ShapeDtypeStruct((n, out_dim), jnp.bfloat16),
            jax.ShapeDtypeStruct((1, out_dim), jnp.float32),
        ],
        scratch_shapes=[
            pltpu.VMEM((n, hid), f8),
            pltpu.VMEM((n, hid), f8),
            pltpu.VMEM((1, hid), jnp.float32),
            pltpu.VMEM((1, 1), jnp.float32),
        ],
        compiler_params=pltpu.CompilerParams(
            dimension_semantics=("arbitrary",),
        ),
    )(gq, p1, w2b)

    out = pl.pallas_call(
        _layer2_kernel,
        grid=(n // bi,),
        in_specs=[
            pl.BlockSpec((bi, n), lambda i: (i, 0)),
            pl.BlockSpec((n, out_dim), lambda i: (0, 0)),
        ],
        out_specs=pl.BlockSpec((bi, out_dim), lambda i: (i, 0)),
        out_shape=jax.ShapeDtypeStruct((n, out_dim), jnp.float32),
        scratch_shapes=[
            pltpu.VMEM((n, out_dim), f8),
            pltpu.VMEM((n, out_dim), f8),
            pltpu.VMEM((1, out_dim), jnp.float32),
            pltpu.VMEM((1, 1), jnp.float32),
        ],
        compiler_params=pltpu.CompilerParams(
            dimension_semantics=("arbitrary",),
        ),
    )(gq, p2)

    return out


# speedup vs baseline: 1.0619x; 1.0619x over previous
import jax
import jax.numpy as jnp
from jax.experimental import pallas as pl
from jax.experimental.pallas import tpu as pltpu


def _feat_kernel(x_ref, w_ref, o_ref, col_ref):
    p = jnp.dot(
        x_ref[...], w_ref[...], preferred_element_type=jnp.float32
    ).astype(jnp.bfloat16)
    o_ref[...] = p
    part = jnp.sum(p.astype(jnp.float32), axis=0, keepdims=True)

    @pl.when(pl.program_id(0) == 0)
    def _():
        col_ref[...] = jnp.zeros_like(col_ref)

    col_ref[...] += part


def _layer0_kernel(g_ref, p_ref, pcol_ref, w_ref, gq_ref, o_ref, col_ref):
    t = g_ref[...] - 0.5
    gq_ref[...] = t.astype(jnp.float8_e4m3fn)
    h = jnp.dot(
        t.astype(jnp.bfloat16), p_ref[...], preferred_element_type=jnp.float32
    )
    h = h + 0.5 * pcol_ref[...]
    h = jnp.maximum(h, 0.0).astype(jnp.bfloat16)
    p = jnp.dot(
        h, w_ref[...], preferred_element_type=jnp.float32
    ).astype(jnp.bfloat16)
    o_ref[...] = p
    part = jnp.sum(p.astype(jnp.float32), axis=0, keepdims=True)

    @pl.when(pl.program_id(0) == 0)
    def _():
        col_ref[...] = jnp.zeros_like(col_ref)

    col_ref[...] += part


def _quant_hilo(p_ref, pq_ref, pcol_ref, s_ref):
    f = p_ref.shape[1]
    p = p_ref[...].astype(jnp.float32)
    s = 440.0 / jnp.maximum(jnp.max(jnp.abs(p)), 1e-30)
    ps = p * s
    phi = ps.astype(jnp.float8_e4m3fn)
    plo = ((ps - phi.astype(jnp.float32)) * 16.0).astype(jnp.float8_e4m3fn)
    pq_ref[:, :f] = phi
    pq_ref[:, f:] = plo
    col = jnp.sum(phi.astype(jnp.float32), axis=0, keepdims=True)
    col += jnp.sum(plo.astype(jnp.float32), axis=0, keepdims=True) * (1.0 / 16.0)
    pcol_ref[...] = col * (1.0 / s)
    s_ref[...] = jnp.full((1, 1), s, jnp.float32)


def _spmm_hilo(g_ref, pq_ref, pcol_ref, s_ref):
    f = pq_ref.shape[1] // 2
    s = s_ref[0, 0]
    acc2 = jnp.dot(g_ref[...], pq_ref[...], preferred_element_type=jnp.float32)
    acc = acc2[:, :f] + acc2[:, f:] * (1.0 / 16.0)
    return acc * (1.0 / s) + 0.5 * pcol_ref[...]


def _layer1_kernel(
    g_ref, p_ref, w_ref, o_ref, col_ref, pq_ref, pcol_ref, s_ref
):
    @pl.when(pl.program_id(0) == 0)
    def _():
        _quant_hilo(p_ref, pq_ref, pcol_ref, s_ref)

    h = _spmm_hilo(g_ref, pq_ref, pcol_ref, s_ref)
    h = jnp.maximum(h, 0.0).astype(jnp.bfloat16)
    p = jnp.dot(
        h, w_ref[...], preferred_element_type=jnp.float32
    ).astype(jnp.bfloat16)
    o_ref[...] = p
    part = jnp.sum(p.astype(jnp.float32), axis=0, keepdims=True)

    @pl.when(pl.program_id(0) == 0)
    def _():
        col_ref[...] = jnp.zeros_like(col_ref)

    col_ref[...] += part


def _layer2_kernel(g_ref, p_ref, o_ref, pq_ref, pcol_ref, s_ref):
    @pl.when(pl.program_id(0) == 0)
    def _():
        _quant_hilo(p_ref, pq_ref, pcol_ref, s_ref)

    o_ref[...] = _spmm_hilo(g_ref, pq_ref, pcol_ref, s_ref)


def kernel(g, inputs, W0, W1, W2):
    n, _ = g.shape
    hid = W0.shape[1]
    out_dim = W2.shape[1]

    bi0 = 400 if n % 400 == 0 else 8
    bi = 1000 if n % 1000 == 0 else (400 if n % 400 == 0 else 8)

    w1b = W1.astype(jnp.bfloat16)
    w2b = W2.astype(jnp.bfloat16)
    f8 = jnp.float8_e4m3fn

    p0, col0 = pl.pallas_call(
        _feat_kernel,
        grid=(n // bi,),
        in_specs=[
            pl.BlockSpec((bi, inputs.shape[1]), lambda i: (i, 0)),
            pl.BlockSpec((inputs.shape[1], hid), lambda i: (0, 0)),
        ],
        out_specs=[
            pl.BlockSpec((bi, hid), lambda i: (i, 0)),
            pl.BlockSpec((1, hid), lambda i: (0, 0)),
        ],
        out_shape=[
            jax.ShapeDtypeStruct((n, hid), jnp.bfloat16),
            jax.ShapeDtypeStruct((1, hid), jnp.float32),
        ],
        compiler_params=pltpu.CompilerParams(
            dimension_semantics=("arbitrary",),
        ),
    )(inputs, W0)

    gq, p1, _col1 = pl.pallas_call(
        _layer0_kernel,
        grid=(n // bi0,),
        in_specs=[
            pl.BlockSpec((bi0, n), lambda i: (i, 0)),
            pl.BlockSpec((n, hid), lambda i: (0, 0)),
            pl.BlockSpec((1, hid), lambda i: (0, 0)),
            pl.BlockSpec((hid, hid), lambda i: (0, 0)),
        ],
        out_specs=[
            pl.BlockSpec((bi0, n), lambda i: (i, 0)),
            pl.BlockSpec((bi0, hid), lambda i: (i, 0)),
            pl.BlockSpec((1, hid), lambda i: (0, 0)),
        ],
        out_shape=[
            jax.ShapeDtypeStruct((n, n), f8),
            jax.ShapeDtypeStruct((n, hid), jnp.bfloat16),
            jax.ShapeDtypeStruct((1, hid), jnp.float32),
        ],
        compiler_params=pltpu.CompilerParams(
            dimension_semantics=("arbitrary",),
        ),
    )(g, p0, col0, w1b)

    p2, _col2 = pl.pallas_call(
        _layer1_kernel,
        grid=(n // bi,),
        in_specs=[
            pl.BlockSpec((bi, n), lambda i: (i, 0)),
            pl.BlockSpec((n, hid), lambda i: (0, 0)),
            pl.BlockSpec((hid, out_dim), lambda i: (0, 0)),
        ],
        out_specs=[
            pl.BlockSpec((bi, out_dim), lambda i: (i, 0)),
            pl.BlockSpec((1, out_dim), lambda i: (0, 0)),
        ],
        out_shape=[
            jax.ShapeDtypeStruct((n, out_dim), jnp.bfloat16),
            jax.ShapeDtypeStruct((1, out_dim), jnp.float32),
        ],
        scratch_shapes=[
            pltpu.VMEM((n, 2 * hid), f8),
            pltpu.VMEM((1, hid), jnp.float32),
            pltpu.VMEM((1, 1), jnp.float32),
        ],
        compiler_params=pltpu.CompilerParams(
            dimension_semantics=("arbitrary",),
        ),
    )(gq, p1, w2b)

    out = pl.pallas_call(
        _layer2_kernel,
        grid=(n // bi,),
        in_specs=[
            pl.BlockSpec((bi, n), lambda i: (i, 0)),
            pl.BlockSpec((n, out_dim), lambda i: (0, 0)),
        ],
        out_specs=pl.BlockSpec((bi, out_dim), lambda i: (i, 0)),
        out_shape=jax.ShapeDtypeStruct((n, out_dim), jnp.float32),
        scratch_shapes=[
            pltpu.VMEM((n, 2 * out_dim), f8),
            pltpu.VMEM((1, out_dim), jnp.float32),
            pltpu.VMEM((1, 1), jnp.float32),
        ],
        compiler_params=pltpu.CompilerParams(
            dimension_semantics=("arbitrary",),
        ),
    )(gq, p2)

    return out


# speedup vs baseline: 1.1820x; 1.1131x over previous
import jax
import jax.numpy as jnp
from jax.experimental import pallas as pl
from jax.experimental.pallas import tpu as pltpu


def _feat_kernel(x_ref, w_ref, o_ref, col_ref):
    p = jnp.dot(
        x_ref[...], w_ref[...], preferred_element_type=jnp.float32
    ).astype(jnp.bfloat16)
    o_ref[...] = p
    col_ref[...] = jnp.sum(p.astype(jnp.float32), axis=0, keepdims=True)


def _layer0_kernel(g_ref, p_ref, pcol_ref, w_ref, gq_ref, o_ref, col_ref):
    t = g_ref[...] - 0.5
    gq_ref[...] = t.astype(jnp.float8_e4m3fn)
    h = jnp.dot(
        t.astype(jnp.bfloat16), p_ref[...], preferred_element_type=jnp.float32
    )
    h = h + 0.5 * pcol_ref[...]
    h = jnp.maximum(h, 0.0).astype(jnp.bfloat16)
    p = jnp.dot(
        h, w_ref[...], preferred_element_type=jnp.float32
    ).astype(jnp.bfloat16)
    o_ref[...] = p
    part = jnp.sum(p.astype(jnp.float32), axis=0, keepdims=True)

    @pl.when(pl.program_id(0) == 0)
    def _():
        col_ref[...] = jnp.zeros_like(col_ref)

    col_ref[...] += part


def _quant_hilo(p_ref, pq_ref, pcol_ref, s_ref):
    f = p_ref.shape[1]
    p = p_ref[...].astype(jnp.float32)
    s = 440.0 / jnp.maximum(jnp.max(jnp.abs(p)), 1e-30)
    ps = p * s
    phi = ps.astype(jnp.float8_e4m3fn)
    plo = ((ps - phi.astype(jnp.float32)) * 16.0).astype(jnp.float8_e4m3fn)
    pq_ref[:, :f] = phi
    pq_ref[:, f:] = plo
    col = jnp.sum(phi.astype(jnp.float32), axis=0, keepdims=True)
    col += jnp.sum(plo.astype(jnp.float32), axis=0, keepdims=True) * (1.0 / 16.0)
    pcol_ref[...] = col * (1.0 / s)
    s_ref[...] = jnp.full((1, 1), s, jnp.float32)


def _spmm_hilo(g_ref, pq_ref, pcol_ref, s_ref):
    f = pq_ref.shape[1] // 2
    s = s_ref[0, 0]
    acc2 = jnp.dot(g_ref[...], pq_ref[...], preferred_element_type=jnp.float32)
    acc = acc2[:, :f] + acc2[:, f:] * (1.0 / 16.0)
    return acc * (1.0 / s) + 0.5 * pcol_ref[...]


def _layer1_kernel(
    g_ref, p_ref, w_ref, o_ref, col_ref, pq_ref, pcol_ref, s_ref
):
    @pl.when(pl.program_id(0) == 0)
    def _():
        p = p_ref[...].astype(jnp.float32)
        s = 440.0 / jnp.maximum(jnp.max(jnp.abs(p)), 1e-30)
        pq = (p * s).astype(jnp.float8_e4m3fn)
        pq_ref[...] = pq
        pcol_ref[...] = jnp.sum(
            pq.astype(jnp.float32), axis=0, keepdims=True
        ) * (1.0 / s)
        s_ref[...] = jnp.full((1, 1), s, jnp.float32)

    s = s_ref[0, 0]
    acc = jnp.dot(g_ref[...], pq_ref[...], preferred_element_type=jnp.float32)
    h = acc * (1.0 / s) + 0.5 * pcol_ref[...]
    h = jnp.maximum(h, 0.0).astype(jnp.bfloat16)
    p = jnp.dot(
        h, w_ref[...], preferred_element_type=jnp.float32
    ).astype(jnp.bfloat16)
    o_ref[...] = p
    part = jnp.sum(p.astype(jnp.float32), axis=0, keepdims=True)

    @pl.when(pl.program_id(0) == 0)
    def _():
        col_ref[...] = jnp.zeros_like(col_ref)

    col_ref[...] += part


def _layer2_kernel(g_ref, p_ref, o_ref, pq_ref, pcol_ref, s_ref):
    @pl.when(pl.program_id(0) == 0)
    def _():
        _quant_hilo(p_ref, pq_ref, pcol_ref, s_ref)

    o_ref[...] = _spmm_hilo(g_ref, pq_ref, pcol_ref, s_ref)


def kernel(g, inputs, W0, W1, W2):
    n, _ = g.shape
    hid = W0.shape[1]
    out_dim = W2.shape[1]

    bi0 = 400 if n % 400 == 0 else 8
    bi = 1000 if n % 1000 == 0 else (400 if n % 400 == 0 else 8)

    w1b = W1.astype(jnp.bfloat16)
    w2b = W2.astype(jnp.bfloat16)
    f8 = jnp.float8_e4m3fn

    p0, col0 = pl.pallas_call(
        _feat_kernel,
        out_shape=[
            jax.ShapeDtypeStruct((n, hid), jnp.bfloat16),
            jax.ShapeDtypeStruct((1, hid), jnp.float32),
        ],
    )(inputs, W0)

    gq, p1, _col1 = pl.pallas_call(
        _layer0_kernel,
        grid=(n // bi0,),
        in_specs=[
            pl.BlockSpec((bi0, n), lambda i: (i, 0)),
            pl.BlockSpec((n, hid), lambda i: (0, 0)),
            pl.BlockSpec((1, hid), lambda i: (0, 0)),
            pl.BlockSpec((hid, hid), lambda i: (0, 0)),
        ],
        out_specs=[
            pl.BlockSpec((bi0, n), lambda i: (i, 0)),
            pl.BlockSpec((bi0, hid), lambda i: (i, 0)),
            pl.BlockSpec((1, hid), lambda i: (0, 0)),
        ],
        out_shape=[
            jax.ShapeDtypeStruct((n, n), f8),
            jax.ShapeDtypeStruct((n, hid), jnp.bfloat16),
            jax.ShapeDtypeStruct((1, hid), jnp.float32),
        ],
        compiler_params=pltpu.CompilerParams(
            dimension_semantics=("arbitrary",),
        ),
    )(g, p0, col0, w1b)

    p2, _col2 = pl.pallas_call(
        _layer1_kernel,
        grid=(n // bi,),
        in_specs=[
            pl.BlockSpec((bi, n), lambda i: (i, 0)),
            pl.BlockSpec((n, hid), lambda i: (0, 0)),
            pl.BlockSpec((hid, out_dim), lambda i: (0, 0)),
        ],
        out_specs=[
            pl.BlockSpec((bi, out_dim), lambda i: (i, 0)),
            pl.BlockSpec((1, out_dim), lambda i: (0, 0)),
        ],
        out_shape=[
            jax.ShapeDtypeStruct((n, out_dim), jnp.bfloat16),
            jax.ShapeDtypeStruct((1, out_dim), jnp.float32),
        ],
        scratch_shapes=[
            pltpu.VMEM((n, hid), f8),
            pltpu.VMEM((1, hid), jnp.float32),
            pltpu.VMEM((1, 1), jnp.float32),
        ],
        compiler_params=pltpu.CompilerParams(
            dimension_semantics=("arbitrary",),
        ),
    )(gq, p1, w2b)

    out = pl.pallas_call(
        _layer2_kernel,
        grid=(n // bi,),
        in_specs=[
            pl.BlockSpec((bi, n), lambda i: (i, 0)),
            pl.BlockSpec((n, out_dim), lambda i: (0, 0)),
        ],
        out_specs=pl.BlockSpec((bi, out_dim), lambda i: (i, 0)),
        out_shape=jax.ShapeDtypeStruct((n, out_dim), jnp.float32),
        scratch_shapes=[
            pltpu.VMEM((n, 2 * out_dim), f8),
            pltpu.VMEM((1, out_dim), jnp.float32),
            pltpu.VMEM((1, 1), jnp.float32),
        ],
        compiler_params=pltpu.CompilerParams(
            dimension_semantics=("arbitrary",),
        ),
    )(gq, p2)

    return out
